# baseline (speedup 1.0000x reference)
import functools

import jax
import jax.numpy as jnp
from jax import lax
from jax.experimental import pallas as pl
from jax.experimental.pallas import tpu as pltpu
from jax.experimental.pallas import tpu_sc as plsc


def kernel(nodes, table):
    B, = nodes.shape
    V, D = table.shape
    nodes32 = nodes.astype(jnp.int32)
    tableT = table.T

    info = plsc.get_sparse_core_info()
    NC, NS, L = info.num_cores, info.num_subcores, info.num_lanes
    NW = NC * NS
    assert D % NW == 0
    d_per_w = D // NW
    assert d_per_w == 2

    VH0 = 49920
    VH1 = V - VH0

    mesh = plsc.VectorSubcoreMesh(core_axis_name="c", subcore_axis_name="s")

    @functools.partial(
        pl.kernel,
        mesh=mesh,
        out_type=jax.ShapeDtypeStruct((D, B), jnp.float32),
        scratch_types=[
            pltpu.VMEM((B,), jnp.int32),
            pltpu.VMEM((VH0,), jnp.float32),
            pltpu.VMEM((VH1,), jnp.float32),
            pltpu.VMEM((B,), jnp.float32),
            pltpu.VMEM((B,), jnp.float32),
            pltpu.SemaphoreType.DMA,
            pltpu.SemaphoreType.DMA,
            pltpu.SemaphoreType.DMA,
            pltpu.SemaphoreType.DMA,
        ],
        compiler_params=pltpu.CompilerParams(needs_layout_passes=False),
    )
    def gather_k(tableT_hbm, idx_hbm, outT_hbm, idx_v, buf_a, buf_b,
                 out0_v, out1_v, sem_i, sem_a, sem_b, sem_s):
        wid = lax.axis_index("s") * NC + lax.axis_index("c")
        j0 = wid * d_per_w
        j1 = j0 + 1
        cp_idx = pltpu.async_copy(idx_hbm, idx_v, sem_i)
        a0 = pltpu.async_copy(tableT_hbm.at[j0, pl.ds(0, VH0)], buf_a, sem_a)
        b0 = pltpu.async_copy(
            tableT_hbm.at[j0, pl.ds(VH0, VH1)], buf_b, sem_b
        )
        cp_idx.wait()

        def pass0(out_ref):
            def body(c, _):
                iv = idx_v[pl.ds(c * L, L)]
                i0 = jnp.minimum(iv, VH0 - 1)
                out_ref[pl.ds(c * L, L)] = plsc.load_gather(buf_a, [i0])
                return ()

            lax.fori_loop(0, B // L, body, (), unroll=4)

        def pass1(out_ref):
            def body(c, _):
                iv = idx_v[pl.ds(c * L, L)]
                i1 = jnp.maximum(iv - VH0, 0)
                v1 = plsc.load_gather(buf_b, [i1])
                prev = out_ref[pl.ds(c * L, L)]
                out_ref[pl.ds(c * L, L)] = jnp.where(iv >= VH0, v1, prev)
                return ()

            lax.fori_loop(0, B // L, body, (), unroll=4)

        a0.wait()
        pass0(out0_v)
        a1 = pltpu.async_copy(tableT_hbm.at[j1, pl.ds(0, VH0)], buf_a, sem_a)
        b0.wait()
        pass1(out0_v)
        b1 = pltpu.async_copy(
            tableT_hbm.at[j1, pl.ds(VH0, VH1)], buf_b, sem_b
        )
        st0 = pltpu.async_copy(out0_v, outT_hbm.at[j0], sem_s)
        a1.wait()
        pass0(out1_v)
        b1.wait()
        pass1(out1_v)
        st0.wait()
        pltpu.sync_copy(out1_v, outT_hbm.at[j1])

    outT = gather_k(tableT, nodes32)
    return outT.T
